# Initial kernel scaffold; baseline (speedup 1.0000x reference)
import jax
import jax.numpy as jnp
from jax.experimental import pallas as pl


def kernel(x):
    raise NotImplementedError("write your pallas kernel here")


# speedup vs baseline: 5.6252x; 5.6252x over previous
import functools

import numpy as np
import jax
import jax.numpy as jnp
from jax.experimental import pallas as pl

T = 8192
N1 = 128
N2 = 64
F = 64
K = 8
HI = jax.lax.Precision.HIGHEST

_k2 = np.arange(N2, dtype=np.float64)
_n2 = np.arange(N2, dtype=np.float64)
_k1 = np.arange(N1, dtype=np.float64)
_n1 = np.arange(N1, dtype=np.float64)
_t2 = np.arange(N2, dtype=np.float64)
_t1 = np.arange(N1, dtype=np.float64)

_th = 2.0 * np.pi * np.outer(_k2, _n2) / N2
C1 = jnp.asarray(np.cos(_th), dtype=jnp.float32)
S1 = jnp.asarray(-np.sin(_th), dtype=jnp.float32)

_th = 2.0 * np.pi * np.outer(_k2, _n1) / T
TWC = jnp.asarray(np.cos(_th), dtype=jnp.float32)
TWS = jnp.asarray(-np.sin(_th), dtype=jnp.float32)

_th = 2.0 * np.pi * np.outer(_n1, _k1) / N1
MC = jnp.asarray(np.cos(_th), dtype=jnp.float32)
MS = jnp.asarray(-np.sin(_th), dtype=jnp.float32)

_th = 2.0 * np.pi * np.outer(_k1, _t1) / N1
EC = jnp.asarray(np.cos(_th), dtype=jnp.float32)
ES = jnp.asarray(np.sin(_th), dtype=jnp.float32)

_th = 2.0 * np.pi * np.outer(_k2, _t1) / T
TIC = jnp.asarray(np.cos(_th), dtype=jnp.float32)
TIS = jnp.asarray(np.sin(_th), dtype=jnp.float32)

_th = 2.0 * np.pi * np.outer(_t2, _k2) / N2
GC = jnp.asarray(np.cos(_th), dtype=jnp.float32)
GS = jnp.asarray(np.sin(_th), dtype=jnp.float32)


def _dot(a, b):
    return jax.lax.dot_general(a, b, (((1,), (0,)), ((), ())),
                               precision=HI, preferred_element_type=jnp.float32)


def _fft_topk_kernel(x_ref, c1, s1, twc, tws, mc, ms, ec, es, tic, tis,
                     gc, gs, main_ref, seas_ref):
    x = x_ref[0]
    x3 = x.reshape(N2, N1, F)
    x3 = jnp.transpose(x3, (0, 2, 1))
    x2 = x3.reshape(N2, F * N1)

    gr = _dot(c1[...], x2)
    gi = _dot(s1[...], x2)

    twc3 = twc[...].reshape(N2, 1, N1)
    tws3 = tws[...].reshape(N2, 1, N1)
    gr3 = gr.reshape(N2, F, N1)
    gi3 = gi.reshape(N2, F, N1)
    hr = gr3 * twc3 - gi3 * tws3
    hi = gr3 * tws3 + gi3 * twc3

    h2r = hr.reshape(N2 * F, N1)
    h2i = hi.reshape(N2 * F, N1)
    xr = _dot(h2r, mc[...]) - _dot(h2i, ms[...])
    xi = _dot(h2r, ms[...]) + _dot(h2i, mc[...])

    mag = jnp.sqrt(xr * xr + xi * xi).reshape(N2, F, N1)
    idx3 = (64 * jax.lax.broadcasted_iota(jnp.int32, (N2, F, N1), 2)
            + jax.lax.broadcasted_iota(jnp.int32, (N2, F, N1), 0))
    mag = jnp.where(idx3 <= 4096, mag, -1.0)

    msk = jnp.zeros((N2, F, N1), jnp.float32)
    for _ in range(K):
        m = jnp.max(jnp.max(mag, axis=2, keepdims=True), axis=0,
                    keepdims=True)
        cand = jnp.where(mag == m, idx3, 16384)
        sel = jnp.min(jnp.min(cand, axis=2, keepdims=True), axis=0,
                      keepdims=True)
        hit = (idx3 == sel) | (idx3 == 8192 - sel)
        msk = jnp.maximum(msk, jnp.where(hit, 1.0, 0.0))
        mag = jnp.where(idx3 == sel, -1.0, mag)

    m2 = msk.reshape(N2 * F, N1)
    yr = xr * m2
    yi = xi * m2

    pr = _dot(yr, ec[...]) - _dot(yi, es[...])
    pi = _dot(yr, es[...]) + _dot(yi, ec[...])

    tic3 = tic[...].reshape(N2, 1, N1)
    tis3 = tis[...].reshape(N2, 1, N1)
    pr3 = pr.reshape(N2, F, N1)
    pi3 = pi.reshape(N2, F, N1)
    qr = pr3 * tic3 - pi3 * tis3
    qi = pr3 * tis3 + pi3 * tic3

    q2r = qr.reshape(N2, F * N1)
    q2i = qi.reshape(N2, F * N1)
    s2 = _dot(gc[...], q2r) - _dot(gs[...], q2i)
    s3 = s2.reshape(N2, F, N1) * (1.0 / T)
    s3 = jnp.transpose(s3, (0, 2, 1))
    seas = s3.reshape(T, F)

    seas_ref[0] = seas
    main_ref[0] = x - seas


@jax.jit
def kernel(x):
    B = x.shape[0]
    tbl_spec = [pl.BlockSpec(t.shape, lambda b: (0,) * t.ndim)
                for t in (C1, S1, TWC, TWS, MC, MS, EC, ES, TIC, TIS, GC, GS)]
    out = pl.pallas_call(
        _fft_topk_kernel,
        grid=(B,),
        in_specs=[pl.BlockSpec((1, T, F), lambda b: (b, 0, 0))] + tbl_spec,
        out_specs=[pl.BlockSpec((1, T, F), lambda b: (b, 0, 0)),
                   pl.BlockSpec((1, T, F), lambda b: (b, 0, 0))],
        out_shape=[jax.ShapeDtypeStruct((B, T, F), jnp.float32),
                   jax.ShapeDtypeStruct((B, T, F), jnp.float32)],
    )(x, C1, S1, TWC, TWS, MC, MS, EC, ES, TIC, TIS, GC, GS)
    return (out[0], out[1])


# speedup vs baseline: 8.4901x; 1.5093x over previous
import functools

import numpy as np
import jax
import jax.numpy as jnp
from jax.experimental import pallas as pl

T = 8192
N1 = 128
N2 = 64
F = 64
K = 8
HI = jax.lax.Precision.HIGHEST
H3 = jax.lax.Precision.DEFAULT

_k2 = np.arange(N2, dtype=np.float64)
_n2 = np.arange(N2, dtype=np.float64)
_k1 = np.arange(N1, dtype=np.float64)
_n1 = np.arange(N1, dtype=np.float64)
_t2 = np.arange(N2, dtype=np.float64)
_t1 = np.arange(N1, dtype=np.float64)

_th = 2.0 * np.pi * np.outer(_k2, _n2) / N2
C1 = np.cos(_th).astype(np.float32)
S1 = (-np.sin(_th)).astype(np.float32)

_th = 2.0 * np.pi * np.outer(_k2, _n1) / T
TWC = np.cos(_th).astype(np.float32)
TWS = (-np.sin(_th)).astype(np.float32)

_th = 2.0 * np.pi * np.outer(_n1, _k1) / N1
MC = np.cos(_th).astype(np.float32)
MS = (-np.sin(_th)).astype(np.float32)

_th = 2.0 * np.pi * np.outer(_k1, _t1) / N1
EC = np.cos(_th).astype(np.float32)
ES = np.sin(_th).astype(np.float32)

_th = 2.0 * np.pi * np.outer(_k2, _t1) / T
TIC = np.cos(_th).astype(np.float32)
TIS = np.sin(_th).astype(np.float32)

_th = 2.0 * np.pi * np.outer(_t2, _k2) / N2
GC = np.cos(_th).astype(np.float32)
GS = np.sin(_th).astype(np.float32)


def _dot(a, b, prec=HI):
    return jax.lax.dot_general(a, b, (((1,), (0,)), ((), ())),
                               precision=prec, preferred_element_type=jnp.float32)


def _fft_topk_kernel(x_ref, c1, s1, twc, tws, mc, ms, ec, es, tic, tis,
                     gc, gs, main_ref, seas_ref):
    x = x_ref[0]
    x3 = x.reshape(N2, N1, F)
    x3 = jnp.transpose(x3, (0, 2, 1))
    x2 = x3.reshape(N2, F * N1)

    gr = _dot(c1[...], x2)
    gi = _dot(s1[...], x2)

    twc3 = twc[...].reshape(N2, 1, N1)
    tws3 = tws[...].reshape(N2, 1, N1)
    gr3 = gr.reshape(N2, F, N1)
    gi3 = gi.reshape(N2, F, N1)
    hr = gr3 * twc3 - gi3 * tws3
    hi = gr3 * tws3 + gi3 * twc3

    h2r = hr.reshape(N2 * F, N1)
    h2i = hi.reshape(N2 * F, N1)
    xr = _dot(h2r, mc[...]) - _dot(h2i, ms[...])
    xi = _dot(h2r, ms[...]) + _dot(h2i, mc[...])

    xrh = xr[:, :64]
    xih = xi[:, :64]
    magh = jnp.sqrt(xrh * xrh + xih * xih).reshape(N2, F, 64)
    nyq_m = jnp.sqrt(xr[0:F, 64:65] ** 2
                     + xi[0:F, 64:65] ** 2).reshape(1, F, 1)
    idxh = (64 * jax.lax.broadcasted_iota(jnp.int32, (N2, F, 64), 2)
            + jax.lax.broadcasted_iota(jnp.int32, (N2, F, 64), 0))

    mskh = jnp.zeros((N2, F, 64), jnp.float32)
    nyq_used = jnp.zeros((1, F, 1), jnp.float32)
    sels = []
    for _ in range(K):
        eff = jnp.where(mskh > 0.0, -1.0, magh)
        mh = jnp.max(jnp.max(eff, axis=2, keepdims=True), axis=0,
                     keepdims=True)
        nyq_eff = jnp.where(nyq_used > 0.0, -1.0, nyq_m)
        m = jnp.maximum(mh, nyq_eff)
        cand = jnp.where(eff == m, idxh, 16384)
        selh = jnp.min(jnp.min(cand, axis=2, keepdims=True), axis=0,
                       keepdims=True)
        sel = jnp.where(nyq_eff == m, jnp.minimum(selh, 4096), selh)
        mskh = jnp.maximum(mskh, jnp.where(idxh == sel, 1.0, 0.0))
        nyq_used = jnp.where(sel == 4096, 1.0, nyq_used)
        sels.append(sel)

    idxu = idxh + 64 * 64
    msku = jnp.zeros((N2, F, 64), jnp.float32)
    for sel in sels:
        msku = jnp.maximum(msku, jnp.where(idxu == 8192 - sel, 1.0, 0.0))

    m2 = jnp.concatenate([mskh.reshape(N2 * F, 64),
                          msku.reshape(N2 * F, 64)], axis=1)
    yr = xr * m2
    yi = xi * m2

    pr = _dot(yr, ec[...], H3) - _dot(yi, es[...], H3)
    pi = _dot(yr, es[...], H3) + _dot(yi, ec[...], H3)

    tic3 = tic[...].reshape(N2, 1, N1)
    tis3 = tis[...].reshape(N2, 1, N1)
    pr3 = pr.reshape(N2, F, N1)
    pi3 = pi.reshape(N2, F, N1)
    qr = pr3 * tic3 - pi3 * tis3
    qi = pr3 * tis3 + pi3 * tic3

    q2r = qr.reshape(N2, F * N1)
    q2i = qi.reshape(N2, F * N1)
    s2 = _dot(gc[...], q2r, H3) - _dot(gs[...], q2i, H3)
    s3 = s2.reshape(N2, F, N1) * (1.0 / T)
    s3 = jnp.transpose(s3, (0, 2, 1))
    seas = s3.reshape(T, F)

    seas_ref[0] = seas
    main_ref[0] = x - seas


@jax.jit
def kernel(x):
    B = x.shape[0]
    tbl_spec = [pl.BlockSpec(t.shape, lambda b: (0,) * t.ndim)
                for t in (C1, S1, TWC, TWS, MC, MS, EC, ES, TIC, TIS, GC, GS)]
    out = pl.pallas_call(
        _fft_topk_kernel,
        grid=(B,),
        in_specs=[pl.BlockSpec((1, T, F), lambda b: (b, 0, 0))] + tbl_spec,
        out_specs=[pl.BlockSpec((1, T, F), lambda b: (b, 0, 0)),
                   pl.BlockSpec((1, T, F), lambda b: (b, 0, 0))],
        out_shape=[jax.ShapeDtypeStruct((B, T, F), jnp.float32),
                   jax.ShapeDtypeStruct((B, T, F), jnp.float32)],
    )(x, C1, S1, TWC, TWS, MC, MS, EC, ES, TIC, TIS, GC, GS)
    return (out[0], out[1])


# speedup vs baseline: 11.7498x; 1.3839x over previous
import functools

import numpy as np
import jax
import jax.numpy as jnp
from jax.experimental import pallas as pl

T = 8192
N1 = 128
N2 = 64
F = 64
K = 8
HI = jax.lax.Precision.HIGHEST
H3 = jax.lax.Precision.DEFAULT

_k2 = np.arange(N2, dtype=np.float64)
_n2 = np.arange(N2, dtype=np.float64)
_k1 = np.arange(N1, dtype=np.float64)
_n1 = np.arange(N1, dtype=np.float64)
_t2 = np.arange(N2, dtype=np.float64)
_t1 = np.arange(N1, dtype=np.float64)

_th = 2.0 * np.pi * np.outer(_k2, _n2) / N2
C1 = np.cos(_th).astype(np.float32)
S1 = (-np.sin(_th)).astype(np.float32)

_th = 2.0 * np.pi * np.outer(_k2, _n1) / T
TWC = np.cos(_th).astype(np.float32)
TWS = (-np.sin(_th)).astype(np.float32)

_th = 2.0 * np.pi * np.outer(_n1, _k1) / N1
MC = np.cos(_th).astype(np.float32)
MS = (-np.sin(_th)).astype(np.float32)

_th = 2.0 * np.pi * np.outer(_k1, _t1) / N1
EC = np.cos(_th).astype(np.float32)
ES = np.sin(_th).astype(np.float32)

_th = 2.0 * np.pi * np.outer(_k2, _t1) / T
TIC = np.cos(_th).astype(np.float32)
TIS = np.sin(_th).astype(np.float32)

_th = 2.0 * np.pi * np.outer(_t2, _k2) / N2
GC = np.cos(_th).astype(np.float32)
GS = np.sin(_th).astype(np.float32)

C1S1 = np.vstack([C1, S1])
MBIG = np.block([[MC, MS], [-MS, MC]])
EBIG = np.block([[EC, ES], [-ES, EC]])
GBIG = np.hstack([GC, -GS])


def _dot(a, b, prec=HI):
    return jax.lax.dot_general(a, b, (((1,), (0,)), ((), ())),
                               precision=prec, preferred_element_type=jnp.float32)


def _fft_topk_kernel(x_ref, c1s1, twc, tws, mbig, ebig, tic, tis,
                     gbig, main_ref, seas_ref):
    x = x_ref[0]
    x3 = x.reshape(N2, N1, F)
    x3 = jnp.transpose(x3, (0, 2, 1))
    x2 = x3.reshape(N2, F * N1)

    g = _dot(c1s1[...], x2)

    twc3 = twc[...].reshape(N2, 1, N1)
    tws3 = tws[...].reshape(N2, 1, N1)
    gr3 = g[:N2].reshape(N2, F, N1)
    gi3 = g[N2:].reshape(N2, F, N1)
    hr = gr3 * twc3 - gi3 * tws3
    hi = gr3 * tws3 + gi3 * twc3

    h2 = jnp.concatenate([hr.reshape(N2 * F, N1),
                          hi.reshape(N2 * F, N1)], axis=1)
    x2d = _dot(h2, mbig[...])
    xr = x2d[:, :N1]
    xi = x2d[:, N1:]

    xrh = xr[:, :64]
    xih = xi[:, :64]
    magh = jnp.sqrt(xrh * xrh + xih * xih).reshape(N2, F, 64)
    nyq_m = jnp.sqrt(xr[0:F, 64:65] ** 2
                     + xi[0:F, 64:65] ** 2).reshape(1, F, 1)
    idxh = (64 * jax.lax.broadcasted_iota(jnp.int32, (N2, F, 64), 2)
            + jax.lax.broadcasted_iota(jnp.int32, (N2, F, 64), 0)
            ).astype(jnp.float32)

    mskh = jnp.zeros((N2, F, 64), jnp.float32)
    nyq_used = jnp.zeros((1, F, 1), jnp.float32)
    sels = []
    for _ in range(K):
        eff = jnp.where(mskh > 0.0, -1.0, magh)
        mh = jnp.max(jnp.max(eff, axis=2, keepdims=True), axis=0,
                     keepdims=True)
        nyq_eff = jnp.where(nyq_used > 0.0, -1.0, nyq_m)
        m = jnp.maximum(mh, nyq_eff)
        cand = jnp.where(eff == m, idxh, 16384.0)
        selh = jnp.min(jnp.min(cand, axis=2, keepdims=True), axis=0,
                       keepdims=True)
        sel = jnp.where(nyq_eff == m, jnp.minimum(selh, 4096.0), selh)
        mskh = jnp.maximum(mskh, jnp.where(idxh == sel, 1.0, 0.0))
        nyq_used = jnp.where(sel == 4096.0, 1.0, nyq_used)
        sels.append(sel)

    idxu = idxh + 4096.0
    msku = jnp.zeros((N2, F, 64), jnp.float32)
    for sel in sels:
        msku = jnp.maximum(msku, jnp.where(idxu == 8192.0 - sel, 1.0, 0.0))

    m2 = jnp.concatenate([mskh.reshape(N2 * F, 64),
                          msku.reshape(N2 * F, 64)], axis=1)
    y2 = x2d * jnp.concatenate([m2, m2], axis=1)

    p2 = _dot(y2, ebig[...], H3)
    pr = p2[:, :N1]
    pi = p2[:, N1:]

    tic3 = tic[...].reshape(N2, 1, N1)
    tis3 = tis[...].reshape(N2, 1, N1)
    pr3 = pr.reshape(N2, F, N1)
    pi3 = pi.reshape(N2, F, N1)
    qr = pr3 * tic3 - pi3 * tis3
    qi = pr3 * tis3 + pi3 * tic3

    qq = jnp.concatenate([qr.reshape(N2, F * N1),
                          qi.reshape(N2, F * N1)], axis=0)
    s2 = _dot(gbig[...], qq, H3)
    s3 = s2.reshape(N2, F, N1) * (1.0 / T)
    s3 = jnp.transpose(s3, (0, 2, 1))
    seas = s3.reshape(T, F)

    seas_ref[0] = seas
    main_ref[0] = x - seas


@jax.jit
def kernel(x):
    B = x.shape[0]
    tbl_spec = [pl.BlockSpec(t.shape, lambda b: (0,) * t.ndim)
                for t in (C1S1, TWC, TWS, MBIG, EBIG, TIC, TIS, GBIG)]
    out = pl.pallas_call(
        _fft_topk_kernel,
        grid=(B,),
        in_specs=[pl.BlockSpec((1, T, F), lambda b: (b, 0, 0))] + tbl_spec,
        out_specs=[pl.BlockSpec((1, T, F), lambda b: (b, 0, 0)),
                   pl.BlockSpec((1, T, F), lambda b: (b, 0, 0))],
        out_shape=[jax.ShapeDtypeStruct((B, T, F), jnp.float32),
                   jax.ShapeDtypeStruct((B, T, F), jnp.float32)],
    )(x, C1S1, TWC, TWS, MBIG, EBIG, TIC, TIS, GBIG)
    return (out[0], out[1])


# speedup vs baseline: 12.3671x; 1.0525x over previous
import functools

import numpy as np
import jax
import jax.numpy as jnp
from jax.experimental import pallas as pl

T = 8192
N1 = 128
N2 = 64
F = 64
K = 8
HI = jax.lax.Precision.HIGHEST
H3 = jax.lax.Precision.DEFAULT

_k2 = np.arange(N2, dtype=np.float64)
_n2 = np.arange(N2, dtype=np.float64)
_k1 = np.arange(N1, dtype=np.float64)
_n1 = np.arange(N1, dtype=np.float64)
_t2 = np.arange(N2, dtype=np.float64)
_t1 = np.arange(N1, dtype=np.float64)

_th = 2.0 * np.pi * np.outer(_k2, _n2) / N2
C1 = np.cos(_th).astype(np.float32)
S1 = (-np.sin(_th)).astype(np.float32)

_th = 2.0 * np.pi * np.outer(_k2, _n1) / T
TWC = np.cos(_th).astype(np.float32)
TWS = (-np.sin(_th)).astype(np.float32)

_th = 2.0 * np.pi * np.outer(_n1, _k1) / N1
MC = np.cos(_th).astype(np.float32)
MS = (-np.sin(_th)).astype(np.float32)

_th = 2.0 * np.pi * np.outer(_k1, _t1) / N1
EC = np.cos(_th).astype(np.float32)
ES = np.sin(_th).astype(np.float32)

_th = 2.0 * np.pi * np.outer(_k2, _t1) / T
TIC = np.cos(_th).astype(np.float32)
TIS = np.sin(_th).astype(np.float32)

_th = 2.0 * np.pi * np.outer(_t2, _k2) / N2
GC = np.cos(_th).astype(np.float32)
GS = np.sin(_th).astype(np.float32)

C1S1 = np.vstack([C1, S1])
MBIG = np.block([[MC, MS], [-MS, MC]])
EBIG = np.block([[EC, ES], [-ES, EC]])
GBIG = np.hstack([GC, -GS])


def _dot(a, b, prec=HI):
    return jax.lax.dot_general(a, b, (((1,), (0,)), ((), ())),
                               precision=prec, preferred_element_type=jnp.float32)


def _fft_topk_kernel(x_ref, c1s1, twc, tws, mbig, ebig, tic, tis,
                     gbig, main_ref, seas_ref):
    x = x_ref[0]
    x3 = x.reshape(N2, N1, F)
    x3 = jnp.transpose(x3, (0, 2, 1))
    x2 = x3.reshape(N2, F * N1)

    g = _dot(c1s1[...], x2)

    twc3 = twc[...].reshape(N2, 1, N1)
    tws3 = tws[...].reshape(N2, 1, N1)
    gr3 = g[:N2].reshape(N2, F, N1)
    gi3 = g[N2:].reshape(N2, F, N1)
    hr = gr3 * twc3 - gi3 * tws3
    hi = gr3 * tws3 + gi3 * twc3

    h2 = jnp.concatenate([hr.reshape(N2 * F, N1),
                          hi.reshape(N2 * F, N1)], axis=1)
    x2d = _dot(h2, mbig[...])
    xr = x2d[:, :N1]
    xi = x2d[:, N1:]

    xrh = xr[:, :64]
    xih = xi[:, :64]
    magh = jnp.sqrt(xrh * xrh + xih * xih).reshape(N2, F, 64)
    nyq_m = jnp.sqrt(xr[0:F, 64:65] ** 2
                     + xi[0:F, 64:65] ** 2).reshape(1, F, 1)
    idxh = (64 * jax.lax.broadcasted_iota(jnp.int32, (N2, F, 64), 2)
            + jax.lax.broadcasted_iota(jnp.int32, (N2, F, 64), 0)
            ).astype(jnp.float32)

    mskh = jnp.zeros((N2, F, 64), jnp.float32)
    nyq_used = jnp.zeros((1, F, 1), jnp.float32)
    sels = []
    for _ in range(K):
        eff = jnp.where(mskh > 0.0, -1.0, magh)
        mh = jnp.max(jnp.max(eff, axis=0, keepdims=True), axis=2,
                     keepdims=True)
        nyq_eff = jnp.where(nyq_used > 0.0, -1.0, nyq_m)
        m = jnp.maximum(mh, nyq_eff)
        cand = jnp.where(eff == m, idxh, 16384.0)
        selh = jnp.min(jnp.min(cand, axis=0, keepdims=True), axis=2,
                       keepdims=True)
        sel = jnp.where(nyq_eff == m, jnp.minimum(selh, 4096.0), selh)
        mskh = jnp.maximum(mskh, jnp.where(idxh == sel, 1.0, 0.0))
        nyq_used = jnp.where(sel == 4096.0, 1.0, nyq_used)
        sels.append(sel)

    idxu = idxh + 4096.0
    msku = jnp.zeros((N2, F, 64), jnp.float32)
    for sel in sels:
        msku = jnp.maximum(msku, jnp.where(idxu == 8192.0 - sel, 1.0, 0.0))

    m2 = jnp.concatenate([mskh.reshape(N2 * F, 64),
                          msku.reshape(N2 * F, 64)], axis=1)
    y2 = x2d * jnp.concatenate([m2, m2], axis=1)

    p2 = _dot(y2, ebig[...], H3)
    pr = p2[:, :N1]
    pi = p2[:, N1:]

    tic3 = tic[...].reshape(N2, 1, N1)
    tis3 = tis[...].reshape(N2, 1, N1)
    pr3 = pr.reshape(N2, F, N1)
    pi3 = pi.reshape(N2, F, N1)
    qr = pr3 * tic3 - pi3 * tis3
    qi = pr3 * tis3 + pi3 * tic3

    qq = jnp.concatenate([qr.reshape(N2, F * N1),
                          qi.reshape(N2, F * N1)], axis=0)
    s2 = _dot(gbig[...], qq, H3)
    s3 = s2.reshape(N2, F, N1) * (1.0 / T)
    s3 = jnp.transpose(s3, (0, 2, 1))
    seas = s3.reshape(T, F)

    seas_ref[0] = seas
    main_ref[0] = x - seas


@jax.jit
def kernel(x):
    B = x.shape[0]
    tbl_spec = [pl.BlockSpec(t.shape, lambda b: (0,) * t.ndim)
                for t in (C1S1, TWC, TWS, MBIG, EBIG, TIC, TIS, GBIG)]
    out = pl.pallas_call(
        _fft_topk_kernel,
        grid=(B,),
        in_specs=[pl.BlockSpec((1, T, F), lambda b: (b, 0, 0))] + tbl_spec,
        out_specs=[pl.BlockSpec((1, T, F), lambda b: (b, 0, 0)),
                   pl.BlockSpec((1, T, F), lambda b: (b, 0, 0))],
        out_shape=[jax.ShapeDtypeStruct((B, T, F), jnp.float32),
                   jax.ShapeDtypeStruct((B, T, F), jnp.float32)],
    )(x, C1S1, TWC, TWS, MBIG, EBIG, TIC, TIS, GBIG)
    return (out[0], out[1])


# speedup vs baseline: 12.3722x; 1.0004x over previous
import functools

import numpy as np
import jax
import jax.numpy as jnp
from jax.experimental import pallas as pl
from jax.experimental.pallas import tpu as pltpu

T = 8192
N1 = 128
N2 = 64
F = 64
K = 8
HI = jax.lax.Precision.HIGHEST
H3 = jax.lax.Precision.DEFAULT

_k2 = np.arange(N2, dtype=np.float64)
_n2 = np.arange(N2, dtype=np.float64)
_k1 = np.arange(N1, dtype=np.float64)
_n1 = np.arange(N1, dtype=np.float64)
_t2 = np.arange(N2, dtype=np.float64)
_t1 = np.arange(N1, dtype=np.float64)

_th = 2.0 * np.pi * np.outer(_k2, _n2) / N2
C1 = np.cos(_th).astype(np.float32)
S1 = (-np.sin(_th)).astype(np.float32)

_th = 2.0 * np.pi * np.outer(_k2, _n1) / T
TWC = np.cos(_th).astype(np.float32)
TWS = (-np.sin(_th)).astype(np.float32)

_th = 2.0 * np.pi * np.outer(_n1, _k1) / N1
MC = np.cos(_th).astype(np.float32)
MS = (-np.sin(_th)).astype(np.float32)

_th = 2.0 * np.pi * np.outer(_k1, _t1) / N1
EC = np.cos(_th).astype(np.float32)
ES = np.sin(_th).astype(np.float32)

_th = 2.0 * np.pi * np.outer(_k2, _t1) / T
TIC = np.cos(_th).astype(np.float32)
TIS = np.sin(_th).astype(np.float32)

_th = 2.0 * np.pi * np.outer(_t2, _k2) / N2
GC = np.cos(_th).astype(np.float32)
GS = np.sin(_th).astype(np.float32)

C1S1 = np.vstack([C1, S1])
MBIG = np.block([[MC, MS], [-MS, MC]])
EBIG = np.block([[EC, ES], [-ES, EC]])
GBIG = np.hstack([GC, -GS])


def _dot(a, b, prec=HI):
    return jax.lax.dot_general(a, b, (((1,), (0,)), ((), ())),
                               precision=prec, preferred_element_type=jnp.float32)


def _fft_topk_kernel(x_ref, c1s1, twc, tws, mbig, ebig, tic, tis,
                     gbig, main_ref, seas_ref):
    x = x_ref[0]
    x3 = x.reshape(N2, N1, F)
    x3 = jnp.transpose(x3, (0, 2, 1))
    x2 = x3.reshape(N2, F * N1)

    g = _dot(c1s1[...], x2)

    twc3 = twc[...].reshape(N2, 1, N1)
    tws3 = tws[...].reshape(N2, 1, N1)
    gr3 = g[:N2].reshape(N2, F, N1)
    gi3 = g[N2:].reshape(N2, F, N1)
    hr = gr3 * twc3 - gi3 * tws3
    hi = gr3 * tws3 + gi3 * twc3

    h2 = jnp.concatenate([hr.reshape(N2 * F, N1),
                          hi.reshape(N2 * F, N1)], axis=1)
    x2d = _dot(h2, mbig[...])
    xr = x2d[:, :N1]
    xi = x2d[:, N1:]

    xrh = xr[:, :64]
    xih = xi[:, :64]
    magh = jnp.sqrt(xrh * xrh + xih * xih).reshape(N2, F, 64)
    nyq_m = jnp.sqrt(xr[0:F, 64:65] ** 2
                     + xi[0:F, 64:65] ** 2).reshape(1, F, 1)
    idxh = (64 * jax.lax.broadcasted_iota(jnp.int32, (N2, F, 64), 2)
            + jax.lax.broadcasted_iota(jnp.int32, (N2, F, 64), 0)
            ).astype(jnp.float32)

    mskh = jnp.zeros((N2, F, 64), jnp.float32)
    nyq_used = jnp.zeros((1, F, 1), jnp.float32)
    sels = []
    for _ in range(K):
        eff = jnp.where(mskh > 0.0, -1.0, magh)
        mh = jnp.max(jnp.max(eff, axis=0, keepdims=True), axis=2,
                     keepdims=True)
        nyq_eff = jnp.where(nyq_used > 0.0, -1.0, nyq_m)
        m = jnp.maximum(mh, nyq_eff)
        cand = jnp.where(eff == m, idxh, 16384.0)
        selh = jnp.min(jnp.min(cand, axis=0, keepdims=True), axis=2,
                       keepdims=True)
        sel = jnp.where(nyq_eff == m, jnp.minimum(selh, 4096.0), selh)
        mskh = jnp.maximum(mskh, jnp.where(idxh == sel, 1.0, 0.0))
        nyq_used = jnp.where(sel == 4096.0, 1.0, nyq_used)
        sels.append(sel)

    idxu = idxh + 4096.0
    msku = jnp.zeros((N2, F, 64), jnp.float32)
    for sel in sels:
        msku = jnp.maximum(msku, jnp.where(idxu == 8192.0 - sel, 1.0, 0.0))

    m2 = jnp.concatenate([mskh.reshape(N2 * F, 64),
                          msku.reshape(N2 * F, 64)], axis=1)
    y2 = x2d * jnp.concatenate([m2, m2], axis=1)

    p2 = _dot(y2, ebig[...], H3)
    pr = p2[:, :N1]
    pi = p2[:, N1:]

    tic3 = tic[...].reshape(N2, 1, N1)
    tis3 = tis[...].reshape(N2, 1, N1)
    pr3 = pr.reshape(N2, F, N1)
    pi3 = pi.reshape(N2, F, N1)
    qr = pr3 * tic3 - pi3 * tis3
    qi = pr3 * tis3 + pi3 * tic3

    qq = jnp.concatenate([qr.reshape(N2, F * N1),
                          qi.reshape(N2, F * N1)], axis=0)
    s2 = _dot(gbig[...], qq, H3)
    s3 = s2.reshape(N2, F, N1) * (1.0 / T)
    s3 = jnp.transpose(s3, (0, 2, 1))
    seas = s3.reshape(T, F)

    seas_ref[0] = seas
    main_ref[0] = x - seas


@jax.jit
def kernel(x):
    B = x.shape[0]
    tbl_spec = [pl.BlockSpec(t.shape, lambda b: (0,) * t.ndim)
                for t in (C1S1, TWC, TWS, MBIG, EBIG, TIC, TIS, GBIG)]
    out = pl.pallas_call(
        _fft_topk_kernel,
        grid=(B,),
        in_specs=[pl.BlockSpec((1, T, F), lambda b: (b, 0, 0))] + tbl_spec,
        out_specs=[pl.BlockSpec((1, T, F), lambda b: (b, 0, 0)),
                   pl.BlockSpec((1, T, F), lambda b: (b, 0, 0))],
        out_shape=[jax.ShapeDtypeStruct((B, T, F), jnp.float32),
                   jax.ShapeDtypeStruct((B, T, F), jnp.float32)],
        compiler_params=pltpu.CompilerParams(
            dimension_semantics=("parallel",)),
    )(x, C1S1, TWC, TWS, MBIG, EBIG, TIC, TIS, GBIG)
    return (out[0], out[1])


# speedup vs baseline: 14.1291x; 1.1420x over previous
import functools

import numpy as np
import jax
import jax.numpy as jnp
from jax.experimental import pallas as pl
from jax.experimental.pallas import tpu as pltpu

T = 8192
N1 = 128
N2 = 64
F = 64
K = 8
HI = jax.lax.Precision.HIGHEST
H3 = jax.lax.Precision.DEFAULT

_k2 = np.arange(N2, dtype=np.float64)
_n2 = np.arange(N2, dtype=np.float64)
_k1 = np.arange(N1, dtype=np.float64)
_n1 = np.arange(N1, dtype=np.float64)
_t2 = np.arange(N2, dtype=np.float64)
_t1 = np.arange(N1, dtype=np.float64)

_th = 2.0 * np.pi * np.outer(_k2, _n2) / N2
C1 = np.cos(_th).astype(np.float32)
S1 = (-np.sin(_th)).astype(np.float32)

_th = 2.0 * np.pi * np.outer(_k2, _n1) / T
TWC = np.cos(_th).astype(np.float32)
TWS = (-np.sin(_th)).astype(np.float32)

_th = 2.0 * np.pi * np.outer(_n1, _k1) / N1
MC = np.cos(_th).astype(np.float32)
MS = (-np.sin(_th)).astype(np.float32)

_th = 2.0 * np.pi * np.outer(_k1, _t1) / N1
EC = np.cos(_th).astype(np.float32)
ES = np.sin(_th).astype(np.float32)

_th = 2.0 * np.pi * np.outer(_k2, _t1) / T
TIC = np.cos(_th).astype(np.float32)
TIS = np.sin(_th).astype(np.float32)

_th = 2.0 * np.pi * np.outer(_t2, _k2) / N2
GC = np.cos(_th).astype(np.float32)
GS = np.sin(_th).astype(np.float32)

C1S1 = np.vstack([C1, S1])
MBIG = np.block([[MC, MS], [-MS, MC]])
EBIG = np.block([[EC, ES], [-ES, EC]])
GBIG = np.hstack([GC, -GS])


def _dot(a, b, prec=HI):
    return jax.lax.dot_general(a, b, (((1,), (0,)), ((), ())),
                               precision=prec, preferred_element_type=jnp.float32)


def _fft_topk_kernel(x_ref, c1s1, twc, tws, mbig, ebig, tic, tis,
                     gbig, main_ref, seas_ref):
    x = x_ref[0]
    x3 = x.reshape(N2, N1, F)
    x3 = jnp.transpose(x3, (0, 2, 1))
    x2 = x3.reshape(N2, F * N1)

    g = _dot(c1s1[...], x2)

    twc3 = twc[...].reshape(N2, 1, N1)
    tws3 = tws[...].reshape(N2, 1, N1)
    gr3 = g[:N2].reshape(N2, F, N1)
    gi3 = g[N2:].reshape(N2, F, N1)
    hr = gr3 * twc3 - gi3 * tws3
    hi = gr3 * tws3 + gi3 * twc3

    h2 = jnp.concatenate([hr.reshape(N2 * F, N1),
                          hi.reshape(N2 * F, N1)], axis=1)
    x2d = _dot(h2, mbig[...])
    xr = x2d[:, :N1]
    xi = x2d[:, N1:]

    xrh = xr[:, :64]
    xih = xi[:, :64]
    magh = jnp.sqrt(xrh * xrh + xih * xih).reshape(N2, F, 64)
    nyq_m = jnp.sqrt(xr[0:F, 64:65] ** 2
                     + xi[0:F, 64:65] ** 2).reshape(1, F, 1)
    idxh = (64 * jax.lax.broadcasted_iota(jnp.int32, (N2, F, 64), 2)
            + jax.lax.broadcasted_iota(jnp.int32, (N2, F, 64), 0)
            ).astype(jnp.float32)

    mskh = jnp.zeros((N2, F, 64), jnp.float32)
    nyq_used = jnp.zeros((1, F, 1), jnp.float32)
    sels = []
    for _ in range(K):
        eff = jnp.where(mskh > 0.0, -1.0, magh)
        mh = jnp.max(jnp.max(eff, axis=0, keepdims=True), axis=2,
                     keepdims=True)
        nyq_eff = jnp.where(nyq_used > 0.0, -1.0, nyq_m)
        m = jnp.maximum(mh, nyq_eff)
        cand = jnp.where(eff == m, idxh, 16384.0)
        selh = jnp.min(jnp.min(cand, axis=0, keepdims=True), axis=2,
                       keepdims=True)
        sel = jnp.where(nyq_eff == m, jnp.minimum(selh, 4096.0), selh)
        mskh = jnp.maximum(mskh, jnp.where(idxh == sel, 1.0, 0.0))
        nyq_used = jnp.where(sel == 4096.0, 1.0, nyq_used)
        sels.append(sel)

    idxu = idxh + 4096.0
    msku = jnp.zeros((N2, F, 64), jnp.float32)
    for sel in sels:
        msku = jnp.maximum(msku, jnp.where(idxu == 8192.0 - sel, 1.0, 0.0))

    m2 = jnp.concatenate([mskh.reshape(N2 * F, 64),
                          msku.reshape(N2 * F, 64)], axis=1)
    y2 = x2d * jnp.concatenate([m2, m2], axis=1)

    p2 = _dot(y2, ebig[...], H3)
    pr = p2[:, :N1]
    pi = p2[:, N1:]

    tic3 = tic[...].reshape(N2, 1, N1)
    tis3 = tis[...].reshape(N2, 1, N1)
    pr3 = pr.reshape(N2, F, N1)
    pi3 = pi.reshape(N2, F, N1)
    qr = pr3 * tic3 - pi3 * tis3
    qi = pr3 * tis3 + pi3 * tic3

    qq = jnp.concatenate([qr.reshape(N2, F * N1),
                          qi.reshape(N2, F * N1)], axis=0)
    s2 = _dot(gbig[...], qq, H3)
    s3 = s2.reshape(N2, F, N1) * (1.0 / T)

    seas_ref[0] = s3
    main_ref[0] = x3 - s3


@jax.jit
def kernel(x):
    B = x.shape[0]
    tbl_spec = [pl.BlockSpec(t.shape, lambda b: (0,) * t.ndim)
                for t in (C1S1, TWC, TWS, MBIG, EBIG, TIC, TIS, GBIG)]
    out = pl.pallas_call(
        _fft_topk_kernel,
        grid=(B,),
        in_specs=[pl.BlockSpec((1, T, F), lambda b: (b, 0, 0))] + tbl_spec,
        out_specs=[pl.BlockSpec((1, N2, F, N1), lambda b: (b, 0, 0, 0)),
                   pl.BlockSpec((1, N2, F, N1), lambda b: (b, 0, 0, 0))],
        out_shape=[jax.ShapeDtypeStruct((B, N2, F, N1), jnp.float32),
                   jax.ShapeDtypeStruct((B, N2, F, N1), jnp.float32)],
        compiler_params=pltpu.CompilerParams(
            dimension_semantics=("parallel",)),
    )(x, C1S1, TWC, TWS, MBIG, EBIG, TIC, TIS, GBIG)
    main = jnp.transpose(out[0], (0, 1, 3, 2)).reshape(B, T, F)
    seas = jnp.transpose(out[1], (0, 1, 3, 2)).reshape(B, T, F)
    return (main, seas)


# speedup vs baseline: 15.5889x; 1.1033x over previous
import functools

import numpy as np
import jax
import jax.numpy as jnp
from jax.experimental import pallas as pl
from jax.experimental.pallas import tpu as pltpu

T = 8192
N1 = 128
N2 = 64
F = 64
K = 8
HI = jax.lax.Precision.HIGHEST
H3 = jax.lax.Precision.DEFAULT

_k2 = np.arange(N2, dtype=np.float64)
_n2 = np.arange(N2, dtype=np.float64)
_k1 = np.arange(N1, dtype=np.float64)
_n1 = np.arange(N1, dtype=np.float64)
_t2 = np.arange(N2, dtype=np.float64)
_t1 = np.arange(N1, dtype=np.float64)

_th = 2.0 * np.pi * np.outer(_k2, _n2) / N2
C1 = np.cos(_th).astype(np.float32)
S1 = (-np.sin(_th)).astype(np.float32)

_th = 2.0 * np.pi * np.outer(_k2, _n1) / T
TWC = np.cos(_th).astype(np.float32)
TWS = (-np.sin(_th)).astype(np.float32)

_th = 2.0 * np.pi * np.outer(_n1, _k1) / N1
MC = np.cos(_th).astype(np.float32)
MS = (-np.sin(_th)).astype(np.float32)

_th = 2.0 * np.pi * np.outer(_k1, _t1) / N1
EC = np.cos(_th).astype(np.float32)
ES = np.sin(_th).astype(np.float32)

_th = 2.0 * np.pi * np.outer(_k2, _t1) / T
TIC = np.cos(_th).astype(np.float32)
TIS = np.sin(_th).astype(np.float32)

_th = 2.0 * np.pi * np.outer(_t2, _k2) / N2
GC = np.cos(_th).astype(np.float32)
GS = np.sin(_th).astype(np.float32)

C1S1 = np.vstack([C1, S1])
MBIG = np.block([[MC, MS], [-MS, MC]])
EBIG = np.block([[EC, ES], [-ES, EC]])
GBIG = np.hstack([GC, -GS])


def _dot(a, b, prec=HI):
    return jax.lax.dot_general(a, b, (((1,), (0,)), ((), ())),
                               precision=prec, preferred_element_type=jnp.float32)


def _fft_topk_kernel(x_ref, c1s1, twc, tws, mbig, ebig, tic, tis,
                     gbig, main_ref, seas_ref):
    x3 = x_ref[0]
    x2 = x3.reshape(N2, F * N1)

    g = _dot(c1s1[...], x2)

    twc3 = twc[...].reshape(N2, 1, N1)
    tws3 = tws[...].reshape(N2, 1, N1)
    gr3 = g[:N2].reshape(N2, F, N1)
    gi3 = g[N2:].reshape(N2, F, N1)
    hr = gr3 * twc3 - gi3 * tws3
    hi = gr3 * tws3 + gi3 * twc3

    h2 = jnp.concatenate([hr.reshape(N2 * F, N1),
                          hi.reshape(N2 * F, N1)], axis=1)
    x2d = _dot(h2, mbig[...])
    xr = x2d[:, :N1]
    xi = x2d[:, N1:]

    xrh = xr[:, :64]
    xih = xi[:, :64]
    magh = jnp.sqrt(xrh * xrh + xih * xih).reshape(N2, F, 64)
    nyq_m = jnp.sqrt(xr[0:F, 64:65] ** 2
                     + xi[0:F, 64:65] ** 2).reshape(1, F, 1)
    idxh = (64 * jax.lax.broadcasted_iota(jnp.int32, (N2, F, 64), 2)
            + jax.lax.broadcasted_iota(jnp.int32, (N2, F, 64), 0)
            ).astype(jnp.float32)

    mskh = jnp.zeros((N2, F, 64), jnp.float32)
    nyq_used = jnp.zeros((1, F, 1), jnp.float32)
    sels = []
    for _ in range(K):
        eff = jnp.where(mskh > 0.0, -1.0, magh)
        mh = jnp.max(jnp.max(eff, axis=0, keepdims=True), axis=2,
                     keepdims=True)
        nyq_eff = jnp.where(nyq_used > 0.0, -1.0, nyq_m)
        m = jnp.maximum(mh, nyq_eff)
        cand = jnp.where(eff == m, idxh, 16384.0)
        selh = jnp.min(jnp.min(cand, axis=0, keepdims=True), axis=2,
                       keepdims=True)
        sel = jnp.where(nyq_eff == m, jnp.minimum(selh, 4096.0), selh)
        mskh = jnp.maximum(mskh, jnp.where(idxh == sel, 1.0, 0.0))
        nyq_used = jnp.where(sel == 4096.0, 1.0, nyq_used)
        sels.append(sel)

    idxu = idxh + 4096.0
    msku = jnp.zeros((N2, F, 64), jnp.float32)
    for sel in sels:
        msku = jnp.maximum(msku, jnp.where(idxu == 8192.0 - sel, 1.0, 0.0))

    m2 = jnp.concatenate([mskh.reshape(N2 * F, 64),
                          msku.reshape(N2 * F, 64)], axis=1)
    y2 = x2d * jnp.concatenate([m2, m2], axis=1)

    p2 = _dot(y2, ebig[...], H3)
    pr = p2[:, :N1]
    pi = p2[:, N1:]

    tic3 = tic[...].reshape(N2, 1, N1)
    tis3 = tis[...].reshape(N2, 1, N1)
    pr3 = pr.reshape(N2, F, N1)
    pi3 = pi.reshape(N2, F, N1)
    qr = pr3 * tic3 - pi3 * tis3
    qi = pr3 * tis3 + pi3 * tic3

    qq = jnp.concatenate([qr.reshape(N2, F * N1),
                          qi.reshape(N2, F * N1)], axis=0)
    s2 = _dot(gbig[...], qq, H3)
    s3 = s2.reshape(N2, F, N1) * (1.0 / T)

    seas_ref[0] = s3
    main_ref[0] = x3 - s3


@jax.jit
def kernel(x):
    B = x.shape[0]
    tbl_spec = [pl.BlockSpec(t.shape, lambda b: (0,) * t.ndim)
                for t in (C1S1, TWC, TWS, MBIG, EBIG, TIC, TIS, GBIG)]
    xp = jnp.transpose(x.reshape(B, N2, N1, F), (0, 1, 3, 2))
    out = pl.pallas_call(
        _fft_topk_kernel,
        grid=(B,),
        in_specs=[pl.BlockSpec((1, N2, F, N1),
                               lambda b: (b, 0, 0, 0))] + tbl_spec,
        out_specs=[pl.BlockSpec((1, N2, F, N1), lambda b: (b, 0, 0, 0)),
                   pl.BlockSpec((1, N2, F, N1), lambda b: (b, 0, 0, 0))],
        out_shape=[jax.ShapeDtypeStruct((B, N2, F, N1), jnp.float32),
                   jax.ShapeDtypeStruct((B, N2, F, N1), jnp.float32)],
        compiler_params=pltpu.CompilerParams(
            dimension_semantics=("parallel",)),
    )(xp, C1S1, TWC, TWS, MBIG, EBIG, TIC, TIS, GBIG)
    main = jnp.transpose(out[0], (0, 1, 3, 2)).reshape(B, T, F)
    seas = jnp.transpose(out[1], (0, 1, 3, 2)).reshape(B, T, F)
    return (main, seas)
